# baseline (speedup 1.0000x reference)
import jax
import jax.numpy as jnp
from jax import lax
from jax.experimental import pallas as pl
from jax.experimental.pallas import tpu as pltpu
from jax.experimental.pallas import tpu_sc as plsc

B = 16384
EMB = 32
NW = 32
CHUNK = 64
NCHUNK = B // CHUNK
CPW = NCHUNK // NW
NSEG = 28
GROWS = NSEG * CHUNK


def _sc_gather_body(bat_t, pit_t, team_t,
                    bat_id, base1, base2, base3, away_sb, home_sb,
                    pit_id, away_pit, home_pit,
                    fld_team, away_team, home_team,
                    x_h, idxA, idxB, gbufA, gbufB,
                    semi, semgA, semgB, semwA, semwB):
    wid = lax.axis_index("s") * 2 + lax.axis_index("c")

    singles = [bat_id, base1, base2, base3]

    def stage_idx(c, idxbuf):
        rows = pl.ds((wid * CPW + c) * CHUNK, CHUNK)

        def idx_dst(s):
            return idxbuf.at[pl.ds(s * CHUNK, CHUNK)]

        for s in range(4):
            pltpu.make_async_copy(singles[s].at[rows], idx_dst(s), semi).start()
        for j in range(9):
            pltpu.make_async_copy(away_sb.at[j, rows], idx_dst(4 + j), semi).start()
            pltpu.make_async_copy(home_sb.at[j, rows], idx_dst(13 + j), semi).start()
        for s, arr in ((22, pit_id), (23, away_pit), (24, home_pit),
                       (25, fld_team), (26, away_team), (27, home_team)):
            pltpu.make_async_copy(arr.at[rows], idx_dst(s), semi).start()
        pltpu.make_async_copy(bat_id.at[pl.ds(0, GROWS)], idxbuf, semi).wait()

    def table(s):
        return bat_t if s < 22 else (pit_t if s < 25 else team_t)

    def fire_gathers(idxbuf, gbuf, semg):
        for s in range(NSEG):
            pltpu.make_async_copy(
                table(s).at[idxbuf.at[pl.ds(s * CHUNK, CHUNK)]],
                gbuf.at[pl.ds(s * CHUNK, CHUNK)], semg).start()

    def drain_by_gbuf(gbuf, sem):
        pltpu.make_async_copy(
            x_h.at[0, pl.ds(0, GROWS), pl.ds(0, EMB)], gbuf, sem).wait()

    def fire_writebacks(c, gbuf, semw):
        rows = pl.ds((wid * CPW + c) * CHUNK, CHUNK)
        for s in range(NSEG):
            pltpu.make_async_copy(
                gbuf.at[pl.ds(s * CHUNK, CHUNK)],
                x_h.at[s // 4, rows, pl.ds((s % 4) * EMB, EMB)], semw).start()

    def pair_body(i, carry):
        c0 = 2 * i
        c1 = 2 * i + 1
        stage_idx(c0, idxA)

        @pl.when(i > 0)
        def _():
            drain_by_gbuf(gbufA, semwA)

        fire_gathers(idxA, gbufA, semgA)
        drain_by_gbuf(gbufA, semgA)
        fire_writebacks(c0, gbufA, semwA)

        stage_idx(c1, idxB)

        @pl.when(i > 0)
        def _():
            drain_by_gbuf(gbufB, semwB)

        fire_gathers(idxB, gbufB, semgB)
        drain_by_gbuf(gbufB, semgB)
        fire_writebacks(c1, gbufB, semwB)
        return carry

    lax.fori_loop(0, CPW // 2, pair_body, 0)
    drain_by_gbuf(gbufA, semwA)
    drain_by_gbuf(gbufB, semwB)


_sc_gather = pl.kernel(
    _sc_gather_body,
    out_type=jax.ShapeDtypeStruct((7, B, 128), jnp.float32),
    mesh=plsc.VectorSubcoreMesh(
        core_axis_name="c", subcore_axis_name="s",
        num_cores=2, num_subcores=16),
    scratch_types=[
        pltpu.VMEM((GROWS,), jnp.int32),
        pltpu.VMEM((GROWS,), jnp.int32),
        pltpu.VMEM((GROWS, EMB), jnp.float32),
        pltpu.VMEM((GROWS, EMB), jnp.float32),
        pltpu.SemaphoreType.DMA,
        pltpu.SemaphoreType.DMA,
        pltpu.SemaphoreType.DMA,
        pltpu.SemaphoreType.DMA,
        pltpu.SemaphoreType.DMA,
    ],
    compiler_params=pltpu.CompilerParams(use_tc_tiling_on_sc=False),
)


def _mlp_body(x, sc, w1, w1s, b1, w2, b2, wh, bh, o0, o1, o2, o3):
    bf16 = jnp.bfloat16
    xb = jnp.concatenate([x[t] for t in range(7)], axis=1).astype(bf16)
    h1 = jnp.dot(xb, w1[...], preferred_element_type=jnp.float32)
    h1 = h1 + jnp.dot(sc[...].T.astype(bf16), w1s[...],
                      preferred_element_type=jnp.float32)
    h1 = jnp.maximum(h1 + b1[...], 0.0).astype(bf16)
    h2 = jnp.maximum(
        jnp.dot(h1, w2[...], preferred_element_type=jnp.float32) + b2[...],
        0.0).astype(bf16)
    lg = jnp.dot(h2, wh[...], preferred_element_type=jnp.float32) + bh[...]
    lgt = lg.T
    for i, o in enumerate((o0, o1, o2, o3)):
        sl = lgt[i * 5:(i + 1) * 5, :]
        m = jnp.max(sl, axis=0, keepdims=True)
        e = jnp.exp(sl - m)
        o[...] = e / jnp.sum(e, axis=0, keepdims=True)


def _mlp_call(BB, x, scal, W1p, W1s, b1r, W2, b2r, Wh, bhm):
    nblk = B // BB
    full = lambda shape: pl.BlockSpec(shape, lambda i: tuple(0 for _ in shape))
    return pl.pallas_call(
        _mlp_body,
        grid=(nblk,),
        in_specs=[
            pl.BlockSpec((7, BB, 128), lambda i: (0, i, 0)),
            pl.BlockSpec((8, BB), lambda i: (0, i)),
            full((896, 512)),
            full((8, 512)),
            full((1, 512)),
            full((512, 256)),
            full((1, 256)),
            full((256, 20)),
            full((1, 20)),
        ],
        out_specs=[pl.BlockSpec((5, BB), lambda i: (0, i))] * 4,
        out_shape=[jax.ShapeDtypeStruct((5, B), jnp.float32)] * 4,
    )(x, scal, W1p, W1s, b1r, W2, b2r, Wh, bhm)


def kernel(outs_ct, bat_id, pit_id, fld_team_id, base1_run_id, base2_run_id,
           base3_run_id, away_score_ct, home_score_ct, inn_ct, bat_home_id,
           away_bat_lineup, home_bat_lineup, away_start_bat_ids,
           home_start_bat_ids, away_pit_id, home_pit_id, away_team_id,
           home_team_id, bat_table, pit_table, team_table, W1, b1, W2, b2,
           Wbd, bbd, Wr1, br1, Wr2, br2, Wr3, br3):
    i32 = jnp.int32
    x = _sc_gather(bat_table, pit_table, team_table,
                   bat_id.astype(i32), base1_run_id.astype(i32),
                   base2_run_id.astype(i32), base3_run_id.astype(i32),
                   away_start_bat_ids.astype(i32).T, home_start_bat_ids.astype(i32).T,
                   pit_id.astype(i32), away_pit_id.astype(i32),
                   home_pit_id.astype(i32),
                   fld_team_id.astype(i32), away_team_id.astype(i32),
                   home_team_id.astype(i32))

    scal = jnp.concatenate(
        [outs_ct.T, away_score_ct.T, home_score_ct.T, inn_ct.T, bat_home_id.T,
         away_bat_lineup.T, home_bat_lineup.T,
         jnp.zeros((1, B), jnp.float32)], axis=0)

    bf16 = jnp.bfloat16
    W1p = jnp.concatenate(
        [W1[1:33], W1[97:193], W1[199:775],
         W1[33:65], W1[775:839],
         W1[65:97], W1[839:903]],
        axis=0).astype(bf16)
    W1s = jnp.concatenate([W1[0:1], W1[193:199],
                           jnp.zeros((1, 512), jnp.float32)],
                          axis=0).astype(bf16)
    Wh = jnp.concatenate([Wbd, Wr1, Wr2, Wr3], axis=1).astype(bf16)
    mask = jnp.array([0.0] * 11 + [-999.0, 0.0, 0.0, 0.0]
                     + [0.0, -999.0, -999.0, 0.0, 0.0], jnp.float32)
    bhm = (jnp.concatenate([bbd, br1, br2, br3]) + mask).reshape(1, 20)

    o0, o1, o2, o3 = _mlp_call(
        4096, x, scal, W1p, W1s,
        b1.reshape(1, 512), W2.astype(bf16), b2.reshape(1, 256), Wh, bhm)
    return (o0.T, o1.T, o2.T, o3.T)


# speedup vs baseline: 1.0021x; 1.0021x over previous
import jax
import jax.numpy as jnp
from jax import lax
from jax.experimental import pallas as pl
from jax.experimental.pallas import tpu as pltpu
from jax.experimental.pallas import tpu_sc as plsc

B = 16384
HALF = B // 2
EMB = 32
NW = 32
CHUNK = 64
NSEG = 28
GROWS = NSEG * CHUNK
CPW = (HALF // CHUNK) // NW


def _sc_gather_body(bat_t, pit_t, team_t,
                    bat_id, base1, base2, base3, away_sb, home_sb,
                    pit_id, away_pit, home_pit,
                    fld_team, away_team, home_team,
                    x_h, idxA, idxB, gbufA, gbufB,
                    semi, semgA, semgB, semwA, semwB):
    wid = lax.axis_index("s") * 2 + lax.axis_index("c")

    singles = [bat_id, base1, base2, base3]

    def stage_idx(c, idxbuf):
        rows = pl.ds((wid * CPW + c) * CHUNK, CHUNK)

        def idx_dst(s):
            return idxbuf.at[pl.ds(s * CHUNK, CHUNK)]

        for s in range(4):
            pltpu.make_async_copy(singles[s].at[rows], idx_dst(s), semi).start()
        for j in range(9):
            pltpu.make_async_copy(away_sb.at[j, rows], idx_dst(4 + j), semi).start()
            pltpu.make_async_copy(home_sb.at[j, rows], idx_dst(13 + j), semi).start()
        for s, arr in ((22, pit_id), (23, away_pit), (24, home_pit),
                       (25, fld_team), (26, away_team), (27, home_team)):
            pltpu.make_async_copy(arr.at[rows], idx_dst(s), semi).start()
        pltpu.make_async_copy(bat_id.at[pl.ds(0, GROWS)], idxbuf, semi).wait()

    def table(s):
        return bat_t if s < 22 else (pit_t if s < 25 else team_t)

    def fire_gathers(idxbuf, gbuf, semg):
        for s in range(NSEG):
            pltpu.make_async_copy(
                table(s).at[idxbuf.at[pl.ds(s * CHUNK, CHUNK)]],
                gbuf.at[pl.ds(s * CHUNK, CHUNK)], semg).start()

    def drain_by_gbuf(gbuf, sem):
        pltpu.make_async_copy(
            x_h.at[0, pl.ds(0, GROWS), pl.ds(0, EMB)], gbuf, sem).wait()

    def fire_writebacks(c, gbuf, semw):
        rows = pl.ds((wid * CPW + c) * CHUNK, CHUNK)
        for s in range(NSEG):
            pltpu.make_async_copy(
                gbuf.at[pl.ds(s * CHUNK, CHUNK)],
                x_h.at[s // 4, rows, pl.ds((s % 4) * EMB, EMB)], semw).start()

    def pair_body(i, carry):
        c0 = 2 * i
        c1 = 2 * i + 1
        stage_idx(c0, idxA)

        @pl.when(i > 0)
        def _():
            drain_by_gbuf(gbufA, semwA)

        fire_gathers(idxA, gbufA, semgA)
        drain_by_gbuf(gbufA, semgA)
        fire_writebacks(c0, gbufA, semwA)

        stage_idx(c1, idxB)

        @pl.when(i > 0)
        def _():
            drain_by_gbuf(gbufB, semwB)

        fire_gathers(idxB, gbufB, semgB)
        drain_by_gbuf(gbufB, semgB)
        fire_writebacks(c1, gbufB, semwB)
        return carry

    lax.fori_loop(0, CPW // 2, pair_body, 0)
    drain_by_gbuf(gbufA, semwA)
    drain_by_gbuf(gbufB, semwB)


_sc_gather_half = pl.kernel(
    _sc_gather_body,
    out_type=jax.ShapeDtypeStruct((7, HALF, 128), jnp.float32),
    mesh=plsc.VectorSubcoreMesh(
        core_axis_name="c", subcore_axis_name="s",
        num_cores=2, num_subcores=16),
    scratch_types=[
        pltpu.VMEM((GROWS,), jnp.int32),
        pltpu.VMEM((GROWS,), jnp.int32),
        pltpu.VMEM((GROWS, EMB), jnp.float32),
        pltpu.VMEM((GROWS, EMB), jnp.float32),
        pltpu.SemaphoreType.DMA,
        pltpu.SemaphoreType.DMA,
        pltpu.SemaphoreType.DMA,
        pltpu.SemaphoreType.DMA,
        pltpu.SemaphoreType.DMA,
    ],
    compiler_params=pltpu.CompilerParams(use_tc_tiling_on_sc=False),
)


def _mlp_body(x, sc, w1, w1s, b1, w2, b2, wh, bh, o0, o1, o2, o3):
    bf16 = jnp.bfloat16
    xb = jnp.concatenate([x[t] for t in range(7)], axis=1).astype(bf16)
    h1 = jnp.dot(xb, w1[...], preferred_element_type=jnp.float32)
    h1 = h1 + jnp.dot(sc[...].T.astype(bf16), w1s[...],
                      preferred_element_type=jnp.float32)
    h1 = jnp.maximum(h1 + b1[...], 0.0).astype(bf16)
    h2 = jnp.maximum(
        jnp.dot(h1, w2[...], preferred_element_type=jnp.float32) + b2[...],
        0.0).astype(bf16)
    lg = jnp.dot(h2, wh[...], preferred_element_type=jnp.float32) + bh[...]
    lgt = lg.T
    for i, o in enumerate((o0, o1, o2, o3)):
        sl = lgt[i * 5:(i + 1) * 5, :]
        m = jnp.max(sl, axis=0, keepdims=True)
        e = jnp.exp(sl - m)
        o[...] = e / jnp.sum(e, axis=0, keepdims=True)


def _mlp_call(BB, x, scal, W1p, W1s, b1r, W2, b2r, Wh, bhm):
    n = x.shape[1]
    nblk = n // BB
    full = lambda shape: pl.BlockSpec(shape, lambda i: tuple(0 for _ in shape))
    return pl.pallas_call(
        _mlp_body,
        grid=(nblk,),
        in_specs=[
            pl.BlockSpec((7, BB, 128), lambda i: (0, i, 0)),
            pl.BlockSpec((8, BB), lambda i: (0, i)),
            full((896, 512)),
            full((8, 512)),
            full((1, 512)),
            full((512, 256)),
            full((1, 256)),
            full((256, 20)),
            full((1, 20)),
        ],
        out_specs=[pl.BlockSpec((5, BB), lambda i: (0, i))] * 4,
        out_shape=[jax.ShapeDtypeStruct((5, n), jnp.float32)] * 4,
    )(x, scal, W1p, W1s, b1r, W2, b2r, Wh, bhm)


def kernel(outs_ct, bat_id, pit_id, fld_team_id, base1_run_id, base2_run_id,
           base3_run_id, away_score_ct, home_score_ct, inn_ct, bat_home_id,
           away_bat_lineup, home_bat_lineup, away_start_bat_ids,
           home_start_bat_ids, away_pit_id, home_pit_id, away_team_id,
           home_team_id, bat_table, pit_table, team_table, W1, b1, W2, b2,
           Wbd, bbd, Wr1, br1, Wr2, br2, Wr3, br3):
    i32 = jnp.int32
    scal_full = jnp.concatenate(
        [outs_ct.T, away_score_ct.T, home_score_ct.T, inn_ct.T, bat_home_id.T,
         away_bat_lineup.T, home_bat_lineup.T,
         jnp.zeros((1, B), jnp.float32)], axis=0)

    bf16 = jnp.bfloat16
    W1p = jnp.concatenate(
        [W1[1:33], W1[97:193], W1[199:775],
         W1[33:65], W1[775:839],
         W1[65:97], W1[839:903]],
        axis=0).astype(bf16)
    W1s = jnp.concatenate([W1[0:1], W1[193:199],
                           jnp.zeros((1, 512), jnp.float32)],
                          axis=0).astype(bf16)
    Wh = jnp.concatenate([Wbd, Wr1, Wr2, Wr3], axis=1).astype(bf16)
    mask = jnp.array([0.0] * 11 + [-999.0, 0.0, 0.0, 0.0]
                     + [0.0, -999.0, -999.0, 0.0, 0.0], jnp.float32)
    bhm = (jnp.concatenate([bbd, br1, br2, br3]) + mask).reshape(1, 20)
    b1r = b1.reshape(1, 512)
    b2r = b2.reshape(1, 256)
    W2b = W2.astype(bf16)

    away_t = away_start_bat_ids.astype(i32).T
    home_t = home_start_bat_ids.astype(i32).T

    outs = []
    for h in range(2):
        sl = slice(h * HALF, (h + 1) * HALF)
        x = _sc_gather_half(
            bat_table, pit_table, team_table,
            bat_id[sl].astype(i32), base1_run_id[sl].astype(i32),
            base2_run_id[sl].astype(i32), base3_run_id[sl].astype(i32),
            away_t[:, sl], home_t[:, sl],
            pit_id[sl].astype(i32), away_pit_id[sl].astype(i32),
            home_pit_id[sl].astype(i32),
            fld_team_id[sl].astype(i32), away_team_id[sl].astype(i32),
            home_team_id[sl].astype(i32))
        outs.append(_mlp_call(2048, x, scal_full[:, sl], W1p, W1s,
                              b1r, W2b, b2r, Wh, bhm))

    return tuple(
        jnp.concatenate([outs[0][k], outs[1][k]], axis=1).T for k in range(4))


# speedup vs baseline: 1.0063x; 1.0042x over previous
import jax
import jax.numpy as jnp
from jax import lax
from jax.experimental import pallas as pl
from jax.experimental.pallas import tpu as pltpu
from jax.experimental.pallas import tpu_sc as plsc

B = 16384
EMB = 32
NW = 32
CHUNK = 64
NCHUNK = B // CHUNK
CPW = NCHUNK // NW
NSEG = 28
GROWS = NSEG * CHUNK


def _sc_gather_body(bat_t, pit_t, team_t,
                    bat_id, base1, base2, base3, away_sb, home_sb,
                    pit_id, away_pit, home_pit,
                    fld_team, away_team, home_team,
                    x_h, idxA, idxB, gbufA, gbufB,
                    semi, semgA, semgB, semwA, semwB):
    wid = lax.axis_index("s") * 2 + lax.axis_index("c")

    singles = [bat_id, base1, base2, base3]

    def stage_idx(c, idxbuf):
        rows = pl.ds((wid * CPW + c) * CHUNK, CHUNK)

        def idx_dst(s):
            return idxbuf.at[pl.ds(s * CHUNK, CHUNK)]

        for s in range(4):
            pltpu.make_async_copy(singles[s].at[rows], idx_dst(s), semi).start()
        for j in range(9):
            pltpu.make_async_copy(away_sb.at[j, rows], idx_dst(4 + j), semi).start()
            pltpu.make_async_copy(home_sb.at[j, rows], idx_dst(13 + j), semi).start()
        for s, arr in ((22, pit_id), (23, away_pit), (24, home_pit),
                       (25, fld_team), (26, away_team), (27, home_team)):
            pltpu.make_async_copy(arr.at[rows], idx_dst(s), semi).start()
        pltpu.make_async_copy(bat_id.at[pl.ds(0, GROWS)], idxbuf, semi).wait()

    def table(s):
        return bat_t if s < 22 else (pit_t if s < 25 else team_t)

    def fire_gathers(idxbuf, gbuf, semg):
        for s in range(NSEG):
            pltpu.make_async_copy(
                table(s).at[idxbuf.at[pl.ds(s * CHUNK, CHUNK)]],
                gbuf.at[pl.ds(s * CHUNK, CHUNK)], semg).start()

    def drain_by_gbuf(gbuf, sem):
        pltpu.make_async_copy(
            x_h.at[0, pl.ds(0, GROWS), pl.ds(0, EMB)], gbuf, sem).wait()

    def fire_writebacks(c, gbuf, semw):
        rows = pl.ds((wid * CPW + c) * CHUNK, CHUNK)
        for s in range(NSEG):
            pltpu.make_async_copy(
                gbuf.at[pl.ds(s * CHUNK, CHUNK)],
                x_h.at[s // 4, rows, pl.ds((s % 4) * EMB, EMB)], semw).start()

    def pair_body(i, carry):
        c0 = 2 * i
        c1 = 2 * i + 1
        stage_idx(c0, idxA)

        @pl.when(i > 0)
        def _():
            drain_by_gbuf(gbufA, semwA)

        fire_gathers(idxA, gbufA, semgA)
        drain_by_gbuf(gbufA, semgA)
        fire_writebacks(c0, gbufA, semwA)

        stage_idx(c1, idxB)

        @pl.when(i > 0)
        def _():
            drain_by_gbuf(gbufB, semwB)

        fire_gathers(idxB, gbufB, semgB)
        drain_by_gbuf(gbufB, semgB)
        fire_writebacks(c1, gbufB, semwB)
        return carry

    lax.fori_loop(0, CPW // 2, pair_body, 0)
    drain_by_gbuf(gbufA, semwA)
    drain_by_gbuf(gbufB, semwB)


_sc_gather = pl.kernel(
    _sc_gather_body,
    out_type=jax.ShapeDtypeStruct((7, B, 128), jnp.float32),
    mesh=plsc.VectorSubcoreMesh(
        core_axis_name="c", subcore_axis_name="s",
        num_cores=2, num_subcores=16),
    scratch_types=[
        pltpu.VMEM((GROWS,), jnp.int32),
        pltpu.VMEM((GROWS,), jnp.int32),
        pltpu.VMEM((GROWS, EMB), jnp.float32),
        pltpu.VMEM((GROWS, EMB), jnp.float32),
        pltpu.SemaphoreType.DMA,
        pltpu.SemaphoreType.DMA,
        pltpu.SemaphoreType.DMA,
        pltpu.SemaphoreType.DMA,
        pltpu.SemaphoreType.DMA,
    ],
    compiler_params=pltpu.CompilerParams(use_tc_tiling_on_sc=False),
)


def _mlp_body(x, sc, w1, w1s, b1, w2, b2, wh, bh, o0, o1, o2, o3):
    bf16 = jnp.bfloat16
    xb = jnp.concatenate([x[t] for t in range(7)], axis=1).astype(bf16)
    h1 = jnp.dot(xb, w1[...], preferred_element_type=jnp.float32)
    h1 = h1 + jnp.dot(sc[...].T.astype(bf16), w1s[...],
                      preferred_element_type=jnp.float32)
    h1 = jnp.maximum(h1 + b1[...], 0.0).astype(bf16)
    h2 = jnp.maximum(
        jnp.dot(h1, w2[...], preferred_element_type=jnp.float32) + b2[...],
        0.0).astype(bf16)
    lg = jnp.dot(h2, wh[...], preferred_element_type=jnp.float32) + bh[...]
    lgt = lg.T
    for i, o in enumerate((o0, o1, o2, o3)):
        sl = lgt[i * 5:(i + 1) * 5, :]
        m = jnp.max(sl, axis=0, keepdims=True)
        e = jnp.exp(sl - m)
        o[...] = e / jnp.sum(e, axis=0, keepdims=True)


def _mlp_call(BB, x, scal, W1p, W1s, b1r, W2, b2r, Wh, bhm):
    nblk = B // BB
    full = lambda shape: pl.BlockSpec(shape, lambda i: tuple(0 for _ in shape))
    return pl.pallas_call(
        _mlp_body,
        grid=(nblk,),
        in_specs=[
            pl.BlockSpec((7, BB, 128), lambda i: (0, i, 0)),
            pl.BlockSpec((8, BB), lambda i: (0, i)),
            full((896, 512)),
            full((8, 512)),
            full((1, 512)),
            full((512, 256)),
            full((1, 256)),
            full((256, 20)),
            full((1, 20)),
        ],
        out_specs=[pl.BlockSpec((5, BB), lambda i: (0, i))] * 4,
        out_shape=[jax.ShapeDtypeStruct((5, B), jnp.float32)] * 4,
    )(x, scal, W1p, W1s, b1r, W2, b2r, Wh, bhm)


def kernel(outs_ct, bat_id, pit_id, fld_team_id, base1_run_id, base2_run_id,
           base3_run_id, away_score_ct, home_score_ct, inn_ct, bat_home_id,
           away_bat_lineup, home_bat_lineup, away_start_bat_ids,
           home_start_bat_ids, away_pit_id, home_pit_id, away_team_id,
           home_team_id, bat_table, pit_table, team_table, W1, b1, W2, b2,
           Wbd, bbd, Wr1, br1, Wr2, br2, Wr3, br3):
    i32 = jnp.int32
    x = _sc_gather(bat_table, pit_table, team_table,
                   bat_id.astype(i32), base1_run_id.astype(i32),
                   base2_run_id.astype(i32), base3_run_id.astype(i32),
                   away_start_bat_ids.astype(i32).T, home_start_bat_ids.astype(i32).T,
                   pit_id.astype(i32), away_pit_id.astype(i32),
                   home_pit_id.astype(i32),
                   fld_team_id.astype(i32), away_team_id.astype(i32),
                   home_team_id.astype(i32))

    scal = jnp.concatenate(
        [outs_ct.T, away_score_ct.T, home_score_ct.T, inn_ct.T, bat_home_id.T,
         away_bat_lineup.T, home_bat_lineup.T,
         jnp.zeros((1, B), jnp.float32)], axis=0)

    bf16 = jnp.bfloat16
    W1p = jnp.concatenate(
        [W1[1:33], W1[97:193], W1[199:775],
         W1[33:65], W1[775:839],
         W1[65:97], W1[839:903]],
        axis=0).astype(bf16)
    W1s = jnp.concatenate([W1[0:1], W1[193:199],
                           jnp.zeros((1, 512), jnp.float32)],
                          axis=0).astype(bf16)
    Wh = jnp.concatenate([Wbd, Wr1, Wr2, Wr3], axis=1).astype(bf16)
    mask = jnp.array([0.0] * 11 + [-999.0, 0.0, 0.0, 0.0]
                     + [0.0, -999.0, -999.0, 0.0, 0.0], jnp.float32)
    bhm = (jnp.concatenate([bbd, br1, br2, br3]) + mask).reshape(1, 20)

    o0, o1, o2, o3 = _mlp_call(
        2048, x, scal, W1p, W1s,
        b1.reshape(1, 512), W2.astype(bf16), b2.reshape(1, 256), Wh, bhm)
    return (o0.T, o1.T, o2.T, o3.T)
